# baseline (speedup 1.0000x reference)
import jax
import jax.numpy as jnp
from jax.experimental import pallas as pl

S, B, W = 20, 1024, 21
V, CE, F = 96, 50, 128
K = 5
T = W - K + 1
TP = 9
NS = 6
N = S * B
NB = 2048
VP = 128


def _fused_body(idx_ref, w3_ref, cb_ref, wp_ref, bp_ref, wg_ref,
                bg_ref, out_ref):
    idx = idx_ref[...].astype(jnp.int16)
    iot = jax.lax.broadcasted_iota(jnp.int16, (W, NB, VP), 2)
    oh = (idx[:, :, None] == iot).astype(jnp.bfloat16)
    ohp = jnp.concatenate(
        [oh, jnp.zeros((1, NB, VP), jnp.bfloat16)], axis=0)
    acc = jnp.zeros((TP * NB, 2 * F), jnp.float32)
    for h in range(3):
        ev = jnp.concatenate(
            [ohp[2 * t2 + 2 * h][None] for t2 in range(TP)], axis=0)
        od = jnp.concatenate(
            [ohp[2 * t2 + 2 * h + 1][None] for t2 in range(TP)], axis=0)
        c = jnp.concatenate([ev.reshape(TP * NB, VP),
                             od.reshape(TP * NB, VP)], axis=1)
        acc = acc + jnp.dot(c, w3_ref[2 * h * VP:(2 * h + 2) * VP, :],
                            preferred_element_type=jnp.float32)
    acc3 = acc.reshape(TP, NB, 2 * F)
    a_even = jnp.max(acc3[:, :, 0:F], axis=0)
    a_odd = jnp.max(acc3[0:TP - 1, :, F:2 * F], axis=0)
    m = jnp.maximum(jnp.maximum(a_even, a_odd) + cb_ref[...], 0.0)
    hp = jnp.maximum(
        jnp.dot(m, wp_ref[...], preferred_element_type=jnp.float32)
        + bp_ref[...], 0.0)
    hg = jax.nn.sigmoid(
        jnp.dot(m, wg_ref[...], preferred_element_type=jnp.float32)
        + bg_ref[...])
    out_ref[...] = hg * hp + (1.0 - hg) * m


def kernel(input, char_emb, conv_w, conv_b, w_proj, b_proj, w_gate, b_gate):
    idxp = input.reshape(N, W).T
    w3 = jnp.einsum('vc,fck->kvf', char_emb, conv_w)
    w3p = jnp.zeros((NS, VP, 2 * F), jnp.float32)
    w3p = w3p.at[:K, :V, 0:F].set(w3)
    w3p = w3p.at[1:K + 1, :V, F:2 * F].set(w3)
    w3p = w3p.reshape(NS * VP, 2 * F).astype(jnp.bfloat16)
    cb2 = conv_b.reshape(1, F)
    bp2 = b_proj.reshape(1, F)
    bg2 = b_gate.reshape(1, F)

    out = pl.pallas_call(
        _fused_body,
        grid=(N // NB,),
        in_specs=[
            pl.BlockSpec((W, NB), lambda i: (0, i)),
            pl.BlockSpec((NS * VP, 2 * F), lambda i: (0, 0)),
            pl.BlockSpec((1, F), lambda i: (0, 0)),
            pl.BlockSpec((F, F), lambda i: (0, 0)),
            pl.BlockSpec((1, F), lambda i: (0, 0)),
            pl.BlockSpec((F, F), lambda i: (0, 0)),
            pl.BlockSpec((1, F), lambda i: (0, 0)),
        ],
        out_specs=pl.BlockSpec((NB, F), lambda i: (i, 0)),
        out_shape=jax.ShapeDtypeStruct((N, F), jnp.float32),
    )(idxp, w3p, cb2, w_proj.T, bp2, w_gate.T, bg2)
    return out.reshape(S, B, F)


# speedup vs baseline: 1.1055x; 1.1055x over previous
import jax
import jax.numpy as jnp
from jax.experimental import pallas as pl

S, B, W = 20, 1024, 21
V, CE, F = 96, 50, 128
K = 5
T = W - K + 1
TP = 9
NS = 6
N = S * B
NB = 2048
VP = 128


def _fused_body(idx_ref, w3_ref, cb_ref, wp_ref, bp_ref, wg_ref,
                bg_ref, out_ref):
    idx = idx_ref[...].astype(jnp.int32)
    iot = jax.lax.broadcasted_iota(jnp.int32, (W, NB, VP), 2)
    oh = (idx[:, :, None] == iot).astype(jnp.bfloat16)
    ohp = jnp.concatenate(
        [oh, jnp.zeros((1, NB, VP), jnp.bfloat16)], axis=0)
    acc = jnp.zeros((TP * NB, 2 * F), jnp.float32)
    for h in range(3):
        ev = jnp.concatenate(
            [ohp[2 * t2 + 2 * h][None] for t2 in range(TP)], axis=0)
        od = jnp.concatenate(
            [ohp[2 * t2 + 2 * h + 1][None] for t2 in range(TP)], axis=0)
        c = jnp.concatenate([ev.reshape(TP * NB, VP),
                             od.reshape(TP * NB, VP)], axis=1)
        acc = acc + jnp.dot(c, w3_ref[2 * h * VP:(2 * h + 2) * VP, :],
                            preferred_element_type=jnp.float32)
    acc3 = acc.reshape(TP, NB, 2 * F)
    a_even = jnp.max(acc3[:, :, 0:F], axis=0)
    a_odd = jnp.max(acc3[0:TP - 1, :, F:2 * F], axis=0)
    m = jnp.maximum(jnp.maximum(a_even, a_odd) + cb_ref[...], 0.0)
    hp = jnp.maximum(
        jnp.dot(m, wp_ref[...], preferred_element_type=jnp.float32)
        + bp_ref[...], 0.0)
    hg = jax.nn.sigmoid(
        jnp.dot(m, wg_ref[...], preferred_element_type=jnp.float32)
        + bg_ref[...])
    out_ref[...] = hg * hp + (1.0 - hg) * m


def kernel(input, char_emb, conv_w, conv_b, w_proj, b_proj, w_gate, b_gate):
    idxp = input.reshape(N, W).astype(jnp.int8).T
    w3 = jnp.einsum('vc,fck->kvf', char_emb, conv_w)
    w3p = jnp.zeros((NS, VP, 2 * F), jnp.float32)
    w3p = w3p.at[:K, :V, 0:F].set(w3)
    w3p = w3p.at[1:K + 1, :V, F:2 * F].set(w3)
    w3p = w3p.reshape(NS * VP, 2 * F).astype(jnp.bfloat16)
    cb2 = conv_b.reshape(1, F)
    bp2 = b_proj.reshape(1, F)
    bg2 = b_gate.reshape(1, F)

    out = pl.pallas_call(
        _fused_body,
        grid=(N // NB,),
        in_specs=[
            pl.BlockSpec((W, NB), lambda i: (0, i)),
            pl.BlockSpec((NS * VP, 2 * F), lambda i: (0, 0)),
            pl.BlockSpec((1, F), lambda i: (0, 0)),
            pl.BlockSpec((F, F), lambda i: (0, 0)),
            pl.BlockSpec((1, F), lambda i: (0, 0)),
            pl.BlockSpec((F, F), lambda i: (0, 0)),
            pl.BlockSpec((1, F), lambda i: (0, 0)),
        ],
        out_specs=pl.BlockSpec((NB, F), lambda i: (i, 0)),
        out_shape=jax.ShapeDtypeStruct((N, F), jnp.float32),
    )(idxp, w3p, cb2, w_proj.T, bp2, w_gate.T, bg2)
    return out.reshape(S, B, F)


# speedup vs baseline: 1.1088x; 1.0029x over previous
import jax
import jax.numpy as jnp
from jax.experimental import pallas as pl

S, B, W = 20, 1024, 21
V, CE, F = 96, 50, 128
K = 5
T = W - K + 1
TP = 9
NS = 6
N = S * B
NB = 2560
VP = 128


def _fused_body(idx_ref, w3_ref, cb_ref, wp_ref, bp_ref, wg_ref,
                bg_ref, out_ref):
    idx = idx_ref[...].astype(jnp.int32)
    iot = jax.lax.broadcasted_iota(jnp.int32, (W, NB, VP), 2)
    oh = (idx[:, :, None] == iot).astype(jnp.bfloat16)
    ohp = jnp.concatenate(
        [oh, jnp.zeros((1, NB, VP), jnp.bfloat16)], axis=0)
    acc = jnp.zeros((TP * NB, 2 * F), jnp.float32)
    for h in range(3):
        ev = jnp.concatenate(
            [ohp[2 * t2 + 2 * h][None] for t2 in range(TP)], axis=0)
        od = jnp.concatenate(
            [ohp[2 * t2 + 2 * h + 1][None] for t2 in range(TP)], axis=0)
        c = jnp.concatenate([ev.reshape(TP * NB, VP),
                             od.reshape(TP * NB, VP)], axis=1)
        acc = acc + jnp.dot(c, w3_ref[2 * h * VP:(2 * h + 2) * VP, :],
                            preferred_element_type=jnp.float32)
    acc3 = acc.reshape(TP, NB, 2 * F)
    a_even = jnp.max(acc3[:, :, 0:F], axis=0)
    a_odd = jnp.max(acc3[0:TP - 1, :, F:2 * F], axis=0)
    m = jnp.maximum(jnp.maximum(a_even, a_odd) + cb_ref[...], 0.0)
    hp = jnp.maximum(
        jnp.dot(m, wp_ref[...], preferred_element_type=jnp.float32)
        + bp_ref[...], 0.0)
    hg = jax.nn.sigmoid(
        jnp.dot(m, wg_ref[...], preferred_element_type=jnp.float32)
        + bg_ref[...])
    out_ref[...] = hg * hp + (1.0 - hg) * m


def kernel(input, char_emb, conv_w, conv_b, w_proj, b_proj, w_gate, b_gate):
    idxp = input.reshape(N, W).astype(jnp.int8).T
    w3 = jnp.einsum('vc,fck->kvf', char_emb, conv_w)
    w3p = jnp.zeros((NS, VP, 2 * F), jnp.float32)
    w3p = w3p.at[:K, :V, 0:F].set(w3)
    w3p = w3p.at[1:K + 1, :V, F:2 * F].set(w3)
    w3p = w3p.reshape(NS * VP, 2 * F).astype(jnp.bfloat16)
    cb2 = conv_b.reshape(1, F)
    bp2 = b_proj.reshape(1, F)
    bg2 = b_gate.reshape(1, F)

    out = pl.pallas_call(
        _fused_body,
        grid=(N // NB,),
        in_specs=[
            pl.BlockSpec((W, NB), lambda i: (0, i)),
            pl.BlockSpec((NS * VP, 2 * F), lambda i: (0, 0)),
            pl.BlockSpec((1, F), lambda i: (0, 0)),
            pl.BlockSpec((F, F), lambda i: (0, 0)),
            pl.BlockSpec((1, F), lambda i: (0, 0)),
            pl.BlockSpec((F, F), lambda i: (0, 0)),
            pl.BlockSpec((1, F), lambda i: (0, 0)),
        ],
        out_specs=pl.BlockSpec((NB, F), lambda i: (i, 0)),
        out_shape=jax.ShapeDtypeStruct((N, F), jnp.float32),
    )(idxp, w3p, cb2, w_proj.T, bp2, w_gate.T, bg2)
    return out.reshape(S, B, F)


# speedup vs baseline: 1.1139x; 1.0046x over previous
import jax
import jax.numpy as jnp
from jax.experimental import pallas as pl

S, B, W = 20, 1024, 21
V, CE, F = 96, 50, 128
K = 5
T = W - K + 1
TP = 9
NS = 6
N = S * B
NB = 4096
VP = 128


def _fused_body(idx_ref, w3_ref, cb_ref, wp_ref, bp_ref, wg_ref,
                bg_ref, out_ref):
    idx = idx_ref[...].astype(jnp.int32)
    iot = jax.lax.broadcasted_iota(jnp.int32, (W, NB, VP), 2)
    oh = (idx[:, :, None] == iot).astype(jnp.bfloat16)
    ohp = jnp.concatenate(
        [oh, jnp.zeros((1, NB, VP), jnp.bfloat16)], axis=0)
    acc = jnp.zeros((TP * NB, 2 * F), jnp.float32)
    for h in range(3):
        ev = jnp.concatenate(
            [ohp[2 * t2 + 2 * h][None] for t2 in range(TP)], axis=0)
        od = jnp.concatenate(
            [ohp[2 * t2 + 2 * h + 1][None] for t2 in range(TP)], axis=0)
        c = jnp.concatenate([ev.reshape(TP * NB, VP),
                             od.reshape(TP * NB, VP)], axis=1)
        acc = acc + jnp.dot(c, w3_ref[2 * h * VP:(2 * h + 2) * VP, :],
                            preferred_element_type=jnp.float32)
    acc3 = acc.reshape(TP, NB, 2 * F)
    a_even = jnp.max(acc3[:, :, 0:F], axis=0)
    a_odd = jnp.max(acc3[0:TP - 1, :, F:2 * F], axis=0)
    m = jnp.maximum(jnp.maximum(a_even, a_odd) + cb_ref[...], 0.0)
    hp = jnp.maximum(
        jnp.dot(m, wp_ref[...], preferred_element_type=jnp.float32)
        + bp_ref[...], 0.0)
    hg = jax.nn.sigmoid(
        jnp.dot(m, wg_ref[...], preferred_element_type=jnp.float32)
        + bg_ref[...])
    out_ref[...] = hg * hp + (1.0 - hg) * m


def kernel(input, char_emb, conv_w, conv_b, w_proj, b_proj, w_gate, b_gate):
    idxp = input.reshape(N, W).astype(jnp.int8).T
    w3 = jnp.einsum('vc,fck->kvf', char_emb, conv_w)
    w3p = jnp.zeros((NS, VP, 2 * F), jnp.float32)
    w3p = w3p.at[:K, :V, 0:F].set(w3)
    w3p = w3p.at[1:K + 1, :V, F:2 * F].set(w3)
    w3p = w3p.reshape(NS * VP, 2 * F).astype(jnp.bfloat16)
    cb2 = conv_b.reshape(1, F)
    bp2 = b_proj.reshape(1, F)
    bg2 = b_gate.reshape(1, F)

    out = pl.pallas_call(
        _fused_body,
        grid=(N // NB,),
        in_specs=[
            pl.BlockSpec((W, NB), lambda i: (0, i)),
            pl.BlockSpec((NS * VP, 2 * F), lambda i: (0, 0)),
            pl.BlockSpec((1, F), lambda i: (0, 0)),
            pl.BlockSpec((F, F), lambda i: (0, 0)),
            pl.BlockSpec((1, F), lambda i: (0, 0)),
            pl.BlockSpec((F, F), lambda i: (0, 0)),
            pl.BlockSpec((1, F), lambda i: (0, 0)),
        ],
        out_specs=pl.BlockSpec((NB, F), lambda i: (i, 0)),
        out_shape=jax.ShapeDtypeStruct((N, F), jnp.float32),
    )(idxp, w3p, cb2, w_proj.T, bp2, w_gate.T, bg2)
    return out.reshape(S, B, F)
